# baseline (speedup 1.0000x reference)
import functools

import jax
import jax.numpy as jnp
from jax import lax
from jax.experimental import pallas as pl
from jax.experimental.pallas import tpu as pltpu
from jax.experimental.pallas import tpu_sc as plsc

N = 25000
N_PAD = 25088
E = 400000
NS = 16
NC = 2
CHUNK = 128
EDGES_PER_TILE = 25088
NCHUNK = EDGES_PER_TILE // CHUNK
SEG = 28
ROWS_PER_TILE = N_PAD // NS
BLK = 1792
GRID = N_PAD // BLK
FBLK = 1000
FGRID = N // FBLK
DIMS = [128, 64, 64, 64, 16]


def _mm4_body(xp_r, xa_r, wsp_r, wnp_r, wsa_r, wna_r,
              sp_o, yp_o, sa_o, ya_o):
    xp = xp_r[...]
    xa = xa_r[...]
    sp_o[...] = jnp.dot(xp, wsp_r[...], preferred_element_type=jnp.float32)
    ya_o[...] = jnp.dot(xp, wna_r[...], preferred_element_type=jnp.float32)
    sa_o[...] = jnp.dot(xa, wsa_r[...], preferred_element_type=jnp.float32)
    yp_o[...] = jnp.dot(xa, wnp_r[...], preferred_element_type=jnp.float32)


def _combine_mm4_body(snp_r, sna_r, bp_r, ba_r, wsp_r, wnp_r, wsa_r, wna_r,
                      sp_o, yp_o, sa_o, ya_o):
    xp = jnp.maximum(snp_r[...] * 0.5 + bp_r[...], 0.0)
    xa = jnp.maximum(sna_r[...] * 0.5 + ba_r[...], 0.0)
    sp_o[...] = jnp.dot(xp, wsp_r[...], preferred_element_type=jnp.float32)
    ya_o[...] = jnp.dot(xp, wna_r[...], preferred_element_type=jnp.float32)
    sa_o[...] = jnp.dot(xa, wsa_r[...], preferred_element_type=jnp.float32)
    yp_o[...] = jnp.dot(xa, wnp_r[...], preferred_element_type=jnp.float32)


def _final_body(snp_r, sna_r, bp_r, ba_r, op_o, oa_o):
    op_o[...] = snp_r[...] * 0.5 + bp_r[...]
    oa_o[...] = sna_r[...] * 0.5 + ba_r[...]


def _mm4(xp, xa, weights, din, dout):
    wsp, wnp, wsa, wna = weights
    bx = pl.BlockSpec((BLK, din), lambda i: (i, 0))
    bw = pl.BlockSpec((din, dout), lambda i: (0, 0))
    bo = pl.BlockSpec((BLK, dout), lambda i: (i, 0))
    return pl.pallas_call(
        _mm4_body,
        grid=(GRID,),
        in_specs=[bx, bx, bw, bw, bw, bw],
        out_specs=[bo, bo, bo, bo],
        out_shape=[jax.ShapeDtypeStruct((N_PAD, dout), jnp.float32)] * 4,
    )(xp, xa, wsp, wnp, wsa, wna)


def _combine_mm4(snp, sna, bp, ba, weights, din, dout):
    wsp, wnp, wsa, wna = weights
    bx = pl.BlockSpec((BLK, din), lambda i: (i, 0))
    bb = pl.BlockSpec((1, din), lambda i: (0, 0))
    bw = pl.BlockSpec((din, dout), lambda i: (0, 0))
    bo = pl.BlockSpec((BLK, dout), lambda i: (i, 0))
    return pl.pallas_call(
        _combine_mm4_body,
        grid=(GRID,),
        in_specs=[bx, bx, bb, bb, bw, bw, bw, bw],
        out_specs=[bo, bo, bo, bo],
        out_shape=[jax.ShapeDtypeStruct((N_PAD, dout), jnp.float32)] * 4,
    )(snp, sna, bp, ba, wsp, wnp, wsa, wna)


def _final_combine(snp, sna, bp, ba, dout):
    bx = pl.BlockSpec((FBLK, dout), lambda i: (i, 0))
    bb = pl.BlockSpec((1, dout), lambda i: (0, 0))
    return pl.pallas_call(
        _final_body,
        grid=(FGRID,),
        in_specs=[bx, bx, bb, bb],
        out_specs=[bx, bx],
        out_shape=[jax.ShapeDtypeStruct((N, dout), jnp.float32)] * 2,
    )(snp, sna, bp, ba)


def _agg_one(y_hbm, s_hbm, src_hbm, dst_hbm, out_hbm,
             src_v, dst_v, r0, r1, sg0, sg1, ss0, ss1, acc):
    sid = lax.axis_index("s")
    base = sid * ROWS_PER_TILE
    pltpu.sync_copy(s_hbm.at[pl.ds(base, ROWS_PER_TILE)],
                    acc.at[pl.ds(base, ROWS_PER_TILE)])
    plsc.subcore_barrier()

    def gwait(buf, sem):
        pltpu.make_async_copy(y_hbm.at[src_v.at[0]], buf, sem).wait()

    def swait(buf, sem):
        pltpu.make_async_copy(buf, acc.at[dst_v.at[0]], sem).wait()

    def seg_body(g, carry):
        pltpu.sync_copy(src_hbm.at[sid, pl.ds(g * SEG, SEG)], src_v)
        pltpu.sync_copy(dst_hbm.at[sid, pl.ds(g * SEG, SEG)], dst_v)
        pltpu.async_copy(y_hbm.at[src_v.at[0]], r0, sg0)
        pltpu.async_copy(y_hbm.at[src_v.at[1]], r1, sg1)

        def pair(k, c):
            j0 = 2 * k
            gwait(r0, sg0)
            pltpu.async_copy(r0, acc.at[dst_v.at[j0]], ss0, add=True)
            gwait(r1, sg1)
            pltpu.async_copy(r1, acc.at[dst_v.at[j0 + 1]], ss1, add=True)

            @pl.when(k < SEG // 2 - 1)
            def _():
                swait(r0, ss0)
                pltpu.async_copy(y_hbm.at[src_v.at[j0 + 2]], r0, sg0)
                swait(r1, ss1)
                pltpu.async_copy(y_hbm.at[src_v.at[j0 + 3]], r1, sg1)

            return c

        lax.fori_loop(0, SEG // 2, pair, carry)
        swait(r0, ss0)
        swait(r1, ss1)
        return carry

    lax.fori_loop(0, NCHUNK // SEG, seg_body, 0)
    plsc.subcore_barrier()
    pltpu.sync_copy(acc.at[pl.ds(base, ROWS_PER_TILE)],
                    out_hbm.at[pl.ds(base, ROWS_PER_TILE)])


def _make_sc_agg(dout):
    mesh = plsc.VectorSubcoreMesh(core_axis_name="c", subcore_axis_name="s",
                                  num_cores=NC, num_subcores=NS)

    @functools.partial(
        pl.kernel,
        out_type=[jax.ShapeDtypeStruct((N_PAD, dout), jnp.float32),
                  jax.ShapeDtypeStruct((N_PAD, dout), jnp.float32)],
        mesh=mesh,
        compiler_params=pltpu.CompilerParams(use_tc_tiling_on_sc=False),
        scratch_types=[
            pltpu.VMEM((SEG, CHUNK), jnp.int32),
            pltpu.VMEM((SEG, CHUNK), jnp.int32),
            pltpu.VMEM((CHUNK, dout), jnp.float32),
            pltpu.VMEM((CHUNK, dout), jnp.float32),
            pltpu.SemaphoreType.DMA,
            pltpu.SemaphoreType.DMA,
            pltpu.SemaphoreType.DMA,
            pltpu.SemaphoreType.DMA,
            pltpu.VMEM_SHARED((N_PAD, dout), jnp.float32),
        ],
    )
    def agg(yp, ya, sp, sa, src_pa, dst_pa, src_ap, dst_ap,
            outp, outa, src_v, dst_v, r0, r1, sg0, sg1, ss0, ss1, acc):
        cid = lax.axis_index("c")

        @pl.when(cid == 0)
        def _():
            _agg_one(yp, sp, src_pa, dst_pa, outp,
                     src_v, dst_v, r0, r1, sg0, sg1, ss0, ss1, acc)

        @pl.when(cid == 1)
        def _():
            _agg_one(ya, sa, src_ap, dst_ap, outa,
                     src_v, dst_v, r0, r1, sg0, sg1, ss0, ss1, acc)

    return agg


_sc_agg = {d: _make_sc_agg(d) for d in (64, 16)}


def _prep_edges(edge):
    pad = NS * EDGES_PER_TILE - E
    src = jnp.pad(edge[1], (0, pad), constant_values=0)
    dst = jnp.pad(edge[0], (0, pad), constant_values=N)
    return (src.reshape(NS, NCHUNK, CHUNK), dst.reshape(NS, NCHUNK, CHUNK))


def kernel(ft_p, ft_a, edge_pa, edge_ap,
           Wself_p1, Wnb_p1, b_p1, Wself_a1, Wnb_a1, b_a1,
           Wself_p2, Wnb_p2, b_p2, Wself_a2, Wnb_a2, b_a2,
           Wself_p3, Wnb_p3, b_p3, Wself_a3, Wnb_a3, b_a3,
           Wself_p4, Wnb_p4, b_p4, Wself_a4, Wnb_a4, b_a4):
    layers = [
        (Wself_p1, Wnb_p1, b_p1, Wself_a1, Wnb_a1, b_a1),
        (Wself_p2, Wnb_p2, b_p2, Wself_a2, Wnb_a2, b_a2),
        (Wself_p3, Wnb_p3, b_p3, Wself_a3, Wnb_a3, b_a3),
        (Wself_p4, Wnb_p4, b_p4, Wself_a4, Wnb_a4, b_a4),
    ]
    xp = jnp.pad(ft_p, ((0, N_PAD - N), (0, 0)))
    xa = jnp.pad(ft_a, ((0, N_PAD - N), (0, 0)))
    src_pa, dst_pa = _prep_edges(edge_pa)
    src_ap, dst_ap = _prep_edges(edge_ap)

    snp = sna = None
    for l, (wsp, wnp, bp, wsa, wna, ba) in enumerate(layers):
        din, dout = DIMS[l], DIMS[l + 1]
        if l == 0:
            sp, yp, sa, ya = _mm4(xp, xa, (wsp, wnp, wsa, wna), din, dout)
        else:
            bp_prev = layers[l - 1][2].reshape(1, din)
            ba_prev = layers[l - 1][5].reshape(1, din)
            sp, yp, sa, ya = _combine_mm4(snp, sna, bp_prev, ba_prev,
                                          (wsp, wnp, wsa, wna), din, dout)
        snp, sna = _sc_agg[dout](yp, ya, sp, sa,
                                 src_pa, dst_pa, src_ap, dst_ap)

    return _final_combine(snp, sna, b_p4.reshape(1, 16), b_a4.reshape(1, 16), 16)


# speedup vs baseline: 1.1119x; 1.1119x over previous
import functools

import jax
import jax.numpy as jnp
from jax import lax
from jax.experimental import pallas as pl
from jax.experimental.pallas import tpu as pltpu
from jax.experimental.pallas import tpu_sc as plsc

N = 25000
N_PAD = 25088
E = 400000
NS = 16
NC = 2
CHUNK = 128
EDGES_PER_TILE = 25088
NCHUNK = EDGES_PER_TILE // CHUNK
SEG = 28
ROWS_PER_TILE = N_PAD // NS
BLK = 1792
GRID = N_PAD // BLK
FBLK = 1000
FGRID = N // FBLK
DIMS = [128, 64, 64, 64, 16]


def _mm4_body(xp_r, xa_r, wsp_r, wnp_r, wsa_r, wna_r,
              sp_o, yp_o, sa_o, ya_o):
    xp = xp_r[...]
    xa = xa_r[...]
    sp_o[...] = jnp.dot(xp, wsp_r[...], preferred_element_type=jnp.float32)
    ya_o[...] = jnp.dot(xp, wna_r[...], preferred_element_type=jnp.float32)
    sa_o[...] = jnp.dot(xa, wsa_r[...], preferred_element_type=jnp.float32)
    yp_o[...] = jnp.dot(xa, wnp_r[...], preferred_element_type=jnp.float32)


def _combine_mm4_body(snp_r, sna_r, bp_r, ba_r, wsp_r, wnp_r, wsa_r, wna_r,
                      sp_o, yp_o, sa_o, ya_o):
    xp = jnp.maximum(snp_r[...] * 0.5 + bp_r[...], 0.0)
    xa = jnp.maximum(sna_r[...] * 0.5 + ba_r[...], 0.0)
    sp_o[...] = jnp.dot(xp, wsp_r[...], preferred_element_type=jnp.float32)
    ya_o[...] = jnp.dot(xp, wna_r[...], preferred_element_type=jnp.float32)
    sa_o[...] = jnp.dot(xa, wsa_r[...], preferred_element_type=jnp.float32)
    yp_o[...] = jnp.dot(xa, wnp_r[...], preferred_element_type=jnp.float32)


def _final_body(snp_r, sna_r, bp_r, ba_r, op_o, oa_o):
    op_o[...] = snp_r[...] * 0.5 + bp_r[...]
    oa_o[...] = sna_r[...] * 0.5 + ba_r[...]


def _mm4(xp, xa, weights, din, dout):
    wsp, wnp, wsa, wna = weights
    bx = pl.BlockSpec((BLK, din), lambda i: (i, 0))
    bw = pl.BlockSpec((din, dout), lambda i: (0, 0))
    bo = pl.BlockSpec((BLK, dout), lambda i: (i, 0))
    return pl.pallas_call(
        _mm4_body,
        grid=(GRID,),
        in_specs=[bx, bx, bw, bw, bw, bw],
        out_specs=[bo, bo, bo, bo],
        out_shape=[jax.ShapeDtypeStruct((N_PAD, dout), jnp.float32)] * 4,
    )(xp, xa, wsp, wnp, wsa, wna)


def _combine_mm4(snp, sna, bp, ba, weights, din, dout):
    wsp, wnp, wsa, wna = weights
    bx = pl.BlockSpec((BLK, din), lambda i: (i, 0))
    bb = pl.BlockSpec((1, din), lambda i: (0, 0))
    bw = pl.BlockSpec((din, dout), lambda i: (0, 0))
    bo = pl.BlockSpec((BLK, dout), lambda i: (i, 0))
    return pl.pallas_call(
        _combine_mm4_body,
        grid=(GRID,),
        in_specs=[bx, bx, bb, bb, bw, bw, bw, bw],
        out_specs=[bo, bo, bo, bo],
        out_shape=[jax.ShapeDtypeStruct((N_PAD, dout), jnp.float32)] * 4,
    )(snp, sna, bp, ba, wsp, wnp, wsa, wna)


def _final_combine(snp, sna, bp, ba, dout):
    bx = pl.BlockSpec((FBLK, dout), lambda i: (i, 0))
    bb = pl.BlockSpec((1, dout), lambda i: (0, 0))
    return pl.pallas_call(
        _final_body,
        grid=(FGRID,),
        in_specs=[bx, bx, bb, bb],
        out_specs=[bx, bx],
        out_shape=[jax.ShapeDtypeStruct((N, dout), jnp.float32)] * 2,
    )(snp, sna, bp, ba)


def _agg_one(y_hbm, s_hbm, src_hbm, dst_hbm, out_hbm,
             src_v, dst_v, r0, r1, sg0, sg1, ss0, ss1, acc):
    sid = lax.axis_index("s")
    base = sid * ROWS_PER_TILE
    pltpu.sync_copy(s_hbm.at[pl.ds(base, ROWS_PER_TILE)],
                    acc.at[pl.ds(base, ROWS_PER_TILE)])
    plsc.subcore_barrier()

    def gwait(buf, sem):
        pltpu.make_async_copy(y_hbm.at[src_v.at[0]], buf, sem).wait()

    def swait(buf, sem):
        pltpu.make_async_copy(buf, acc.at[dst_v.at[0]], sem).wait()

    def seg_body(g, carry):
        pltpu.sync_copy(src_hbm.at[sid, pl.ds(g * SEG, SEG)], src_v)
        pltpu.sync_copy(dst_hbm.at[sid, pl.ds(g * SEG, SEG)], dst_v)
        pltpu.async_copy(y_hbm.at[src_v.at[0]], r0, sg0)
        pltpu.async_copy(y_hbm.at[src_v.at[1]], r1, sg1)

        def pair(k, c):
            j0 = 2 * k
            gwait(r0, sg0)
            pltpu.async_copy(r0, acc.at[dst_v.at[j0]], ss0, add=True)

            @pl.when(k < SEG // 2 - 1)
            def _():
                swait(r0, ss0)
                pltpu.async_copy(y_hbm.at[src_v.at[j0 + 2]], r0, sg0)

            gwait(r1, sg1)
            pltpu.async_copy(r1, acc.at[dst_v.at[j0 + 1]], ss1, add=True)

            @pl.when(k < SEG // 2 - 1)
            def _():
                swait(r1, ss1)
                pltpu.async_copy(y_hbm.at[src_v.at[j0 + 3]], r1, sg1)

            return c

        lax.fori_loop(0, SEG // 2, pair, carry)
        swait(r0, ss0)
        swait(r1, ss1)
        return carry

    lax.fori_loop(0, NCHUNK // SEG, seg_body, 0)
    plsc.subcore_barrier()
    pltpu.sync_copy(acc.at[pl.ds(base, ROWS_PER_TILE)],
                    out_hbm.at[pl.ds(base, ROWS_PER_TILE)])


def _make_sc_agg(dout):
    mesh = plsc.VectorSubcoreMesh(core_axis_name="c", subcore_axis_name="s",
                                  num_cores=NC, num_subcores=NS)

    @functools.partial(
        pl.kernel,
        out_type=[jax.ShapeDtypeStruct((N_PAD, dout), jnp.float32),
                  jax.ShapeDtypeStruct((N_PAD, dout), jnp.float32)],
        mesh=mesh,
        compiler_params=pltpu.CompilerParams(use_tc_tiling_on_sc=False),
        scratch_types=[
            pltpu.VMEM((SEG, CHUNK), jnp.int32),
            pltpu.VMEM((SEG, CHUNK), jnp.int32),
            pltpu.VMEM((CHUNK, dout), jnp.float32),
            pltpu.VMEM((CHUNK, dout), jnp.float32),
            pltpu.SemaphoreType.DMA,
            pltpu.SemaphoreType.DMA,
            pltpu.SemaphoreType.DMA,
            pltpu.SemaphoreType.DMA,
            pltpu.VMEM_SHARED((N_PAD, dout), jnp.float32),
        ],
    )
    def agg(yp, ya, sp, sa, src_pa, dst_pa, src_ap, dst_ap,
            outp, outa, src_v, dst_v, r0, r1, sg0, sg1, ss0, ss1, acc):
        cid = lax.axis_index("c")

        @pl.when(cid == 0)
        def _():
            _agg_one(yp, sp, src_pa, dst_pa, outp,
                     src_v, dst_v, r0, r1, sg0, sg1, ss0, ss1, acc)

        @pl.when(cid == 1)
        def _():
            _agg_one(ya, sa, src_ap, dst_ap, outa,
                     src_v, dst_v, r0, r1, sg0, sg1, ss0, ss1, acc)

    return agg


_sc_agg = {d: _make_sc_agg(d) for d in (64, 16)}


def _prep_edges(edge):
    pad = NS * EDGES_PER_TILE - E
    src = jnp.pad(edge[1], (0, pad), constant_values=0)
    dst = jnp.pad(edge[0], (0, pad), constant_values=N)
    return (src.reshape(NS, NCHUNK, CHUNK), dst.reshape(NS, NCHUNK, CHUNK))


def kernel(ft_p, ft_a, edge_pa, edge_ap,
           Wself_p1, Wnb_p1, b_p1, Wself_a1, Wnb_a1, b_a1,
           Wself_p2, Wnb_p2, b_p2, Wself_a2, Wnb_a2, b_a2,
           Wself_p3, Wnb_p3, b_p3, Wself_a3, Wnb_a3, b_a3,
           Wself_p4, Wnb_p4, b_p4, Wself_a4, Wnb_a4, b_a4):
    layers = [
        (Wself_p1, Wnb_p1, b_p1, Wself_a1, Wnb_a1, b_a1),
        (Wself_p2, Wnb_p2, b_p2, Wself_a2, Wnb_a2, b_a2),
        (Wself_p3, Wnb_p3, b_p3, Wself_a3, Wnb_a3, b_a3),
        (Wself_p4, Wnb_p4, b_p4, Wself_a4, Wnb_a4, b_a4),
    ]
    xp = jnp.pad(ft_p, ((0, N_PAD - N), (0, 0)))
    xa = jnp.pad(ft_a, ((0, N_PAD - N), (0, 0)))
    src_pa, dst_pa = _prep_edges(edge_pa)
    src_ap, dst_ap = _prep_edges(edge_ap)

    snp = sna = None
    for l, (wsp, wnp, bp, wsa, wna, ba) in enumerate(layers):
        din, dout = DIMS[l], DIMS[l + 1]
        if l == 0:
            sp, yp, sa, ya = _mm4(xp, xa, (wsp, wnp, wsa, wna), din, dout)
        else:
            bp_prev = layers[l - 1][2].reshape(1, din)
            ba_prev = layers[l - 1][5].reshape(1, din)
            sp, yp, sa, ya = _combine_mm4(snp, sna, bp_prev, ba_prev,
                                          (wsp, wnp, wsa, wna), din, dout)
        snp, sna = _sc_agg[dout](yp, ya, sp, sa,
                                 src_pa, dst_pa, src_ap, dst_ap)

    return _final_combine(snp, sna, b_p4.reshape(1, 16), b_a4.reshape(1, 16), 16)


# speedup vs baseline: 1.1767x; 1.0583x over previous
import functools

import jax
import jax.numpy as jnp
from jax import lax
from jax.experimental import pallas as pl
from jax.experimental.pallas import tpu as pltpu
from jax.experimental.pallas import tpu_sc as plsc

N = 25000
N_PAD = 25088
E = 400000
NS = 16
NC = 2
CHUNK = 128
EDGES_PER_TILE = 25088
NCHUNK = EDGES_PER_TILE // CHUNK
SEG = 28
ROWS_PER_TILE = N_PAD // NS
BLK = 1792
GRID = N_PAD // BLK
FBLK = 1000
FGRID = N // FBLK
DIMS = [128, 64, 64, 64, 16]


def _mm4_body(xp_r, xa_r, wsp_r, wnp_r, wsa_r, wna_r,
              sp_o, yp_o, sa_o, ya_o):
    xp = xp_r[...]
    xa = xa_r[...]
    sp_o[...] = jnp.dot(xp, wsp_r[...], preferred_element_type=jnp.float32)
    ya_o[...] = jnp.dot(xp, wna_r[...], preferred_element_type=jnp.float32)
    sa_o[...] = jnp.dot(xa, wsa_r[...], preferred_element_type=jnp.float32)
    yp_o[...] = jnp.dot(xa, wnp_r[...], preferred_element_type=jnp.float32)


def _combine_mm4_body(nbp_r, nba_r, sp_r, sa_r, bp_r, ba_r,
                      wsp_r, wnp_r, wsa_r, wna_r,
                      sp_o, yp_o, sa_o, ya_o):
    xp = jnp.maximum((sp_r[...] + nbp_r[...]) * 0.5 + bp_r[...], 0.0)
    xa = jnp.maximum((sa_r[...] + nba_r[...]) * 0.5 + ba_r[...], 0.0)
    sp_o[...] = jnp.dot(xp, wsp_r[...], preferred_element_type=jnp.float32)
    ya_o[...] = jnp.dot(xp, wna_r[...], preferred_element_type=jnp.float32)
    sa_o[...] = jnp.dot(xa, wsa_r[...], preferred_element_type=jnp.float32)
    yp_o[...] = jnp.dot(xa, wnp_r[...], preferred_element_type=jnp.float32)


def _final_body(nbp_r, nba_r, sp_r, sa_r, bp_r, ba_r, op_o, oa_o):
    op_o[...] = (sp_r[...] + nbp_r[...]) * 0.5 + bp_r[...]
    oa_o[...] = (sa_r[...] + nba_r[...]) * 0.5 + ba_r[...]


def _mm4(xp, xa, weights, din, dout):
    wsp, wnp, wsa, wna = weights
    bx = pl.BlockSpec((BLK, din), lambda i: (i, 0))
    bw = pl.BlockSpec((din, dout), lambda i: (0, 0))
    bo = pl.BlockSpec((BLK, dout), lambda i: (i, 0))
    return pl.pallas_call(
        _mm4_body,
        grid=(GRID,),
        in_specs=[bx, bx, bw, bw, bw, bw],
        out_specs=[bo, bo, bo, bo],
        out_shape=[jax.ShapeDtypeStruct((N_PAD, dout), jnp.float32)] * 4,
    )(xp, xa, wsp, wnp, wsa, wna)


def _combine_mm4(nbp, nba, sp, sa, bp, ba, weights, din, dout):
    wsp, wnp, wsa, wna = weights
    bx = pl.BlockSpec((BLK, din), lambda i: (i, 0))
    bb = pl.BlockSpec((1, din), lambda i: (0, 0))
    bw = pl.BlockSpec((din, dout), lambda i: (0, 0))
    bo = pl.BlockSpec((BLK, dout), lambda i: (i, 0))
    return pl.pallas_call(
        _combine_mm4_body,
        grid=(GRID,),
        in_specs=[bx, bx, bx, bx, bb, bb, bw, bw, bw, bw],
        out_specs=[bo, bo, bo, bo],
        out_shape=[jax.ShapeDtypeStruct((N_PAD, dout), jnp.float32)] * 4,
    )(nbp, nba, sp, sa, bp, ba, wsp, wnp, wsa, wna)


def _final_combine(nbp, nba, sp, sa, bp, ba, dout):
    bx = pl.BlockSpec((FBLK, dout), lambda i: (i, 0))
    bb = pl.BlockSpec((1, dout), lambda i: (0, 0))
    return pl.pallas_call(
        _final_body,
        grid=(FGRID,),
        in_specs=[bx, bx, bx, bx, bb, bb],
        out_specs=[bx, bx],
        out_shape=[jax.ShapeDtypeStruct((N, dout), jnp.float32)] * 2,
    )(nbp, nba, sp, sa, bp, ba)


def _agg_one(y_hbm, z_hbm, src_hbm, dst_hbm, out_hbm,
             src_v, dst_v, r0, r1, sg0, sg1, ss0, ss1, acc):
    sid = lax.axis_index("s")
    base = sid * ROWS_PER_TILE
    pltpu.sync_copy(z_hbm, acc.at[pl.ds(base, ROWS_PER_TILE)])
    plsc.subcore_barrier()

    def gwait(buf, sem):
        pltpu.make_async_copy(y_hbm.at[src_v.at[0]], buf, sem).wait()

    def swait(buf, sem):
        pltpu.make_async_copy(buf, acc.at[dst_v.at[0]], sem).wait()

    def seg_body(g, carry):
        pltpu.sync_copy(src_hbm.at[sid, pl.ds(g * SEG, SEG)], src_v)
        pltpu.sync_copy(dst_hbm.at[sid, pl.ds(g * SEG, SEG)], dst_v)
        pltpu.async_copy(y_hbm.at[src_v.at[0]], r0, sg0)
        pltpu.async_copy(y_hbm.at[src_v.at[1]], r1, sg1)

        def pair(k, c):
            j0 = 2 * k
            gwait(r0, sg0)
            pltpu.async_copy(r0, acc.at[dst_v.at[j0]], ss0, add=True)

            @pl.when(k < SEG // 2 - 1)
            def _():
                swait(r0, ss0)
                pltpu.async_copy(y_hbm.at[src_v.at[j0 + 2]], r0, sg0)

            gwait(r1, sg1)
            pltpu.async_copy(r1, acc.at[dst_v.at[j0 + 1]], ss1, add=True)

            @pl.when(k < SEG // 2 - 1)
            def _():
                swait(r1, ss1)
                pltpu.async_copy(y_hbm.at[src_v.at[j0 + 3]], r1, sg1)

            return c

        lax.fori_loop(0, SEG // 2, pair, carry)
        swait(r0, ss0)
        swait(r1, ss1)
        return carry

    lax.fori_loop(0, NCHUNK // SEG, seg_body, 0)
    plsc.subcore_barrier()
    pltpu.sync_copy(acc.at[pl.ds(base, ROWS_PER_TILE)],
                    out_hbm.at[pl.ds(base, ROWS_PER_TILE)])


def _make_sc_agg(dout):
    mesh = plsc.VectorSubcoreMesh(core_axis_name="c", subcore_axis_name="s",
                                  num_cores=NC, num_subcores=NS)

    @functools.partial(
        pl.kernel,
        out_type=[jax.ShapeDtypeStruct((N_PAD, dout), jnp.float32),
                  jax.ShapeDtypeStruct((N_PAD, dout), jnp.float32)],
        mesh=mesh,
        compiler_params=pltpu.CompilerParams(use_tc_tiling_on_sc=False),
        scratch_types=[
            pltpu.VMEM((SEG, CHUNK), jnp.int32),
            pltpu.VMEM((SEG, CHUNK), jnp.int32),
            pltpu.VMEM((CHUNK, dout), jnp.float32),
            pltpu.VMEM((CHUNK, dout), jnp.float32),
            pltpu.SemaphoreType.DMA,
            pltpu.SemaphoreType.DMA,
            pltpu.SemaphoreType.DMA,
            pltpu.SemaphoreType.DMA,
            pltpu.VMEM_SHARED((N_PAD, dout), jnp.float32),
        ],
    )
    def agg(yp, ya, zeros, src_pa, dst_pa, src_ap, dst_ap,
            outp, outa, src_v, dst_v, r0, r1, sg0, sg1, ss0, ss1, acc):
        cid = lax.axis_index("c")

        @pl.when(cid == 0)
        def _():
            _agg_one(yp, zeros, src_pa, dst_pa, outp,
                     src_v, dst_v, r0, r1, sg0, sg1, ss0, ss1, acc)

        @pl.when(cid == 1)
        def _():
            _agg_one(ya, zeros, src_ap, dst_ap, outa,
                     src_v, dst_v, r0, r1, sg0, sg1, ss0, ss1, acc)

    return agg


_sc_agg = {d: _make_sc_agg(d) for d in (64, 16)}


def _prep_edges(edge):
    pad = NS * EDGES_PER_TILE - E
    src = jnp.pad(edge[1], (0, pad), constant_values=0)
    dst = jnp.pad(edge[0], (0, pad), constant_values=N)
    return (src.reshape(NS, NCHUNK, CHUNK), dst.reshape(NS, NCHUNK, CHUNK))


def kernel(ft_p, ft_a, edge_pa, edge_ap,
           Wself_p1, Wnb_p1, b_p1, Wself_a1, Wnb_a1, b_a1,
           Wself_p2, Wnb_p2, b_p2, Wself_a2, Wnb_a2, b_a2,
           Wself_p3, Wnb_p3, b_p3, Wself_a3, Wnb_a3, b_a3,
           Wself_p4, Wnb_p4, b_p4, Wself_a4, Wnb_a4, b_a4):
    layers = [
        (Wself_p1, Wnb_p1, b_p1, Wself_a1, Wnb_a1, b_a1),
        (Wself_p2, Wnb_p2, b_p2, Wself_a2, Wnb_a2, b_a2),
        (Wself_p3, Wnb_p3, b_p3, Wself_a3, Wnb_a3, b_a3),
        (Wself_p4, Wnb_p4, b_p4, Wself_a4, Wnb_a4, b_a4),
    ]
    xp = jnp.pad(ft_p, ((0, N_PAD - N), (0, 0)))
    xa = jnp.pad(ft_a, ((0, N_PAD - N), (0, 0)))
    src_pa, dst_pa = _prep_edges(edge_pa)
    src_ap, dst_ap = _prep_edges(edge_ap)

    nbp = nba = sp = sa = None
    for l, (wsp, wnp, bp, wsa, wna, ba) in enumerate(layers):
        din, dout = DIMS[l], DIMS[l + 1]
        if l == 0:
            sp, yp, sa, ya = _mm4(xp, xa, (wsp, wnp, wsa, wna), din, dout)
        else:
            bp_prev = layers[l - 1][2].reshape(1, din)
            ba_prev = layers[l - 1][5].reshape(1, din)
            sp, yp, sa, ya = _combine_mm4(nbp, nba, sp, sa, bp_prev, ba_prev,
                                          (wsp, wnp, wsa, wna), din, dout)
        zeros = jnp.zeros((ROWS_PER_TILE, dout), jnp.float32)
        nbp, nba = _sc_agg[dout](yp, ya, zeros,
                                 src_pa, dst_pa, src_ap, dst_ap)

    return _final_combine(nbp, nba, sp, sa,
                          b_p4.reshape(1, 16), b_a4.reshape(1, 16), 16)


# speedup vs baseline: 1.2099x; 1.0282x over previous
import functools

import jax
import jax.numpy as jnp
from jax import lax
from jax.experimental import pallas as pl
from jax.experimental.pallas import tpu as pltpu
from jax.experimental.pallas import tpu_sc as plsc

N = 25000
N_PAD = 25088
E = 400000
NS = 16
NC = 2
CHUNK = 128
EDGES_PER_TILE = 25088
NCHUNK = EDGES_PER_TILE // CHUNK
SEG = 14
NBUF = 3
ROWS_PER_TILE = N_PAD // NS
BLK = 1792
GRID = N_PAD // BLK
FBLK = 1000
FGRID = N // FBLK
DIMS = [128, 64, 64, 64, 16]


def _mm4_body(xp_r, xa_r, wsp_r, wnp_r, wsa_r, wna_r,
              sp_o, yp_o, sa_o, ya_o):
    xp = xp_r[...]
    xa = xa_r[...]
    sp_o[...] = jnp.dot(xp, wsp_r[...], preferred_element_type=jnp.float32)
    ya_o[...] = jnp.dot(xp, wna_r[...], preferred_element_type=jnp.float32)
    sa_o[...] = jnp.dot(xa, wsa_r[...], preferred_element_type=jnp.float32)
    yp_o[...] = jnp.dot(xa, wnp_r[...], preferred_element_type=jnp.float32)


def _combine_mm4_body(nbp_r, nba_r, sp_r, sa_r, bp_r, ba_r,
                      wsp_r, wnp_r, wsa_r, wna_r,
                      sp_o, yp_o, sa_o, ya_o):
    xp = jnp.maximum((sp_r[...] + nbp_r[...]) * 0.5 + bp_r[...], 0.0)
    xa = jnp.maximum((sa_r[...] + nba_r[...]) * 0.5 + ba_r[...], 0.0)
    sp_o[...] = jnp.dot(xp, wsp_r[...], preferred_element_type=jnp.float32)
    ya_o[...] = jnp.dot(xp, wna_r[...], preferred_element_type=jnp.float32)
    sa_o[...] = jnp.dot(xa, wsa_r[...], preferred_element_type=jnp.float32)
    yp_o[...] = jnp.dot(xa, wnp_r[...], preferred_element_type=jnp.float32)


def _final_body(nbp_r, nba_r, sp_r, sa_r, bp_r, ba_r, op_o, oa_o):
    op_o[...] = (sp_r[...] + nbp_r[...]) * 0.5 + bp_r[...]
    oa_o[...] = (sa_r[...] + nba_r[...]) * 0.5 + ba_r[...]


def _mm4(xp, xa, weights, din, dout):
    wsp, wnp, wsa, wna = weights
    bx = pl.BlockSpec((BLK, din), lambda i: (i, 0))
    bw = pl.BlockSpec((din, dout), lambda i: (0, 0))
    bo = pl.BlockSpec((BLK, dout), lambda i: (i, 0))
    return pl.pallas_call(
        _mm4_body,
        grid=(GRID,),
        in_specs=[bx, bx, bw, bw, bw, bw],
        out_specs=[bo, bo, bo, bo],
        out_shape=[jax.ShapeDtypeStruct((N_PAD, dout), jnp.float32)] * 4,
    )(xp, xa, wsp, wnp, wsa, wna)


def _combine_mm4(nbp, nba, sp, sa, bp, ba, weights, din, dout):
    wsp, wnp, wsa, wna = weights
    bx = pl.BlockSpec((BLK, din), lambda i: (i, 0))
    bb = pl.BlockSpec((1, din), lambda i: (0, 0))
    bw = pl.BlockSpec((din, dout), lambda i: (0, 0))
    bo = pl.BlockSpec((BLK, dout), lambda i: (i, 0))
    return pl.pallas_call(
        _combine_mm4_body,
        grid=(GRID,),
        in_specs=[bx, bx, bx, bx, bb, bb, bw, bw, bw, bw],
        out_specs=[bo, bo, bo, bo],
        out_shape=[jax.ShapeDtypeStruct((N_PAD, dout), jnp.float32)] * 4,
    )(nbp, nba, sp, sa, bp, ba, wsp, wnp, wsa, wna)


def _final_combine(nbp, nba, sp, sa, bp, ba, dout):
    bx = pl.BlockSpec((FBLK, dout), lambda i: (i, 0))
    bb = pl.BlockSpec((1, dout), lambda i: (0, 0))
    return pl.pallas_call(
        _final_body,
        grid=(FGRID,),
        in_specs=[bx, bx, bx, bx, bb, bb],
        out_specs=[bx, bx],
        out_shape=[jax.ShapeDtypeStruct((N, dout), jnp.float32)] * 2,
    )(nbp, nba, sp, sa, bp, ba)


def _agg_one(y_hbm, z_hbm, src_hbm, dst_hbm, out_hbm,
             src_v, dst_v, bufs, gsems, ssems, acc):
    sid = lax.axis_index("s")
    base = sid * ROWS_PER_TILE
    pltpu.sync_copy(z_hbm, acc.at[pl.ds(base, ROWS_PER_TILE)])
    plsc.subcore_barrier()

    def gwait(b):
        pltpu.make_async_copy(y_hbm.at[src_v.at[0]], bufs[b], gsems[b]).wait()

    def swait(b):
        pltpu.make_async_copy(bufs[b], acc.at[dst_v.at[0]], ssems[b]).wait()

    def seg_body(g, carry):
        pltpu.sync_copy(src_hbm.at[sid, pl.ds(g * SEG, SEG)], src_v)
        pltpu.sync_copy(dst_hbm.at[sid, pl.ds(g * SEG, SEG)], dst_v)
        for j in range(2):
            pltpu.async_copy(y_hbm.at[src_v.at[j]], bufs[j], gsems[j])
        for j in range(SEG):
            b = j % NBUF
            gwait(b)
            pltpu.async_copy(bufs[b], acc.at[dst_v.at[j]], ssems[b], add=True)
            if j + 2 < SEG:
                nb = (j + 2) % NBUF
                if j > 0:
                    swait(nb)
                pltpu.async_copy(y_hbm.at[src_v.at[j + 2]], bufs[nb], gsems[nb])
        for b in range(NBUF):
            swait(b)
        return carry

    lax.fori_loop(0, NCHUNK // SEG, seg_body, 0)
    plsc.subcore_barrier()
    pltpu.sync_copy(acc.at[pl.ds(base, ROWS_PER_TILE)],
                    out_hbm.at[pl.ds(base, ROWS_PER_TILE)])


def _make_sc_agg(dout):
    mesh = plsc.VectorSubcoreMesh(core_axis_name="c", subcore_axis_name="s",
                                  num_cores=NC, num_subcores=NS)

    @functools.partial(
        pl.kernel,
        out_type=[jax.ShapeDtypeStruct((N_PAD, dout), jnp.float32),
                  jax.ShapeDtypeStruct((N_PAD, dout), jnp.float32)],
        mesh=mesh,
        compiler_params=pltpu.CompilerParams(use_tc_tiling_on_sc=False),
        scratch_types=[
            pltpu.VMEM((SEG, CHUNK), jnp.int32),
            pltpu.VMEM((SEG, CHUNK), jnp.int32),
        ] + [pltpu.VMEM((CHUNK, dout), jnp.float32)] * NBUF
          + [pltpu.SemaphoreType.DMA] * (2 * NBUF)
          + [pltpu.VMEM_SHARED((N_PAD, dout), jnp.float32)],
    )
    def agg(yp, ya, zeros, src_pa, dst_pa, src_ap, dst_ap,
            outp, outa, src_v, dst_v, *rest):
        bufs = list(rest[:NBUF])
        gsems = list(rest[NBUF:2 * NBUF])
        ssems = list(rest[2 * NBUF:3 * NBUF])
        acc = rest[3 * NBUF]
        cid = lax.axis_index("c")

        @pl.when(cid == 0)
        def _():
            _agg_one(yp, zeros, src_pa, dst_pa, outp,
                     src_v, dst_v, bufs, gsems, ssems, acc)

        @pl.when(cid == 1)
        def _():
            _agg_one(ya, zeros, src_ap, dst_ap, outa,
                     src_v, dst_v, bufs, gsems, ssems, acc)

    return agg


_sc_agg = {d: _make_sc_agg(d) for d in (64, 16)}


def _prep_edges(edge):
    pad = NS * EDGES_PER_TILE - E
    src = jnp.pad(edge[1], (0, pad), constant_values=0)
    dst = jnp.pad(edge[0], (0, pad), constant_values=N)
    return (src.reshape(NS, NCHUNK, CHUNK), dst.reshape(NS, NCHUNK, CHUNK))


def kernel(ft_p, ft_a, edge_pa, edge_ap,
           Wself_p1, Wnb_p1, b_p1, Wself_a1, Wnb_a1, b_a1,
           Wself_p2, Wnb_p2, b_p2, Wself_a2, Wnb_a2, b_a2,
           Wself_p3, Wnb_p3, b_p3, Wself_a3, Wnb_a3, b_a3,
           Wself_p4, Wnb_p4, b_p4, Wself_a4, Wnb_a4, b_a4):
    layers = [
        (Wself_p1, Wnb_p1, b_p1, Wself_a1, Wnb_a1, b_a1),
        (Wself_p2, Wnb_p2, b_p2, Wself_a2, Wnb_a2, b_a2),
        (Wself_p3, Wnb_p3, b_p3, Wself_a3, Wnb_a3, b_a3),
        (Wself_p4, Wnb_p4, b_p4, Wself_a4, Wnb_a4, b_a4),
    ]
    xp = jnp.pad(ft_p, ((0, N_PAD - N), (0, 0)))
    xa = jnp.pad(ft_a, ((0, N_PAD - N), (0, 0)))
    src_pa, dst_pa = _prep_edges(edge_pa)
    src_ap, dst_ap = _prep_edges(edge_ap)

    nbp = nba = sp = sa = None
    for l, (wsp, wnp, bp, wsa, wna, ba) in enumerate(layers):
        din, dout = DIMS[l], DIMS[l + 1]
        if l == 0:
            sp, yp, sa, ya = _mm4(xp, xa, (wsp, wnp, wsa, wna), din, dout)
        else:
            bp_prev = layers[l - 1][2].reshape(1, din)
            ba_prev = layers[l - 1][5].reshape(1, din)
            sp, yp, sa, ya = _combine_mm4(nbp, nba, sp, sa, bp_prev, ba_prev,
                                          (wsp, wnp, wsa, wna), din, dout)
        zeros = jnp.zeros((ROWS_PER_TILE, dout), jnp.float32)
        nbp, nba = _sc_agg[dout](yp, ya, zeros,
                                 src_pa, dst_pa, src_ap, dst_ap)

    return _final_combine(nbp, nba, sp, sa,
                          b_p4.reshape(1, 16), b_a4.reshape(1, 16), 16)


# speedup vs baseline: 1.2888x; 1.0652x over previous
import functools

import jax
import jax.numpy as jnp
from jax import lax
from jax.experimental import pallas as pl
from jax.experimental.pallas import tpu as pltpu
from jax.experimental.pallas import tpu_sc as plsc

N = 25000
N_PAD = 25088
E = 400000
NS = 16
NC = 2
CHUNK = 128
EDGES_PER_TILE = 25088
NCHUNK = EDGES_PER_TILE // CHUNK
SEG = 14
NBUF = 3
ROWS_PER_TILE = N_PAD // NS
BLK = 1792
GRID = N_PAD // BLK
FBLK = 1000
FGRID = N // FBLK
DIMS = [128, 64, 64, 64, 16]


def _mm4_body(xp_r, xa_r, wsp_r, wnp_r, wsa_r, wna_r,
              sp_o, yp_o, sa_o, ya_o):
    xp = xp_r[...]
    xa = xa_r[...]
    sp_o[...] = jnp.dot(xp, wsp_r[...], preferred_element_type=jnp.float32)
    ya_o[...] = jnp.dot(xp, wna_r[...], preferred_element_type=jnp.float32)
    sa_o[...] = jnp.dot(xa, wsa_r[...], preferred_element_type=jnp.float32)
    yp_o[...] = jnp.dot(xa, wnp_r[...], preferred_element_type=jnp.float32)


def _combine_mm4_body(nbp_r, nba_r, sp_r, sa_r, bp_r, ba_r,
                      wsp_r, wnp_r, wsa_r, wna_r,
                      sp_o, yp_o, sa_o, ya_o):
    din = sp_r.shape[1]
    xp = jnp.maximum((sp_r[...] + nbp_r[:, :din]) * 0.5 + bp_r[...], 0.0)
    xa = jnp.maximum((sa_r[...] + nba_r[:, :din]) * 0.5 + ba_r[...], 0.0)
    sp_o[...] = jnp.dot(xp, wsp_r[...], preferred_element_type=jnp.float32)
    ya_o[...] = jnp.dot(xp, wna_r[...], preferred_element_type=jnp.float32)
    sa_o[...] = jnp.dot(xa, wsa_r[...], preferred_element_type=jnp.float32)
    yp_o[...] = jnp.dot(xa, wnp_r[...], preferred_element_type=jnp.float32)


def _final_body(nbp_r, nba_r, sp_r, sa_r, bp_r, ba_r, op_o, oa_o):
    dout = sp_r.shape[1]
    op_o[...] = (sp_r[...] + nbp_r[:, :dout]) * 0.5 + bp_r[...]
    oa_o[...] = (sa_r[...] + nba_r[:, :dout]) * 0.5 + ba_r[...]


def _mm4(xp, xa, weights, din, dout):
    wsp, wnp, wsa, wna = weights
    bx = pl.BlockSpec((BLK, din), lambda i: (i, 0))
    bw = pl.BlockSpec((din, dout), lambda i: (0, 0))
    bo = pl.BlockSpec((BLK, dout), lambda i: (i, 0))
    return pl.pallas_call(
        _mm4_body,
        grid=(GRID,),
        in_specs=[bx, bx, bw, bw, bw, bw],
        out_specs=[bo, bo, bo, bo],
        out_shape=[jax.ShapeDtypeStruct((N_PAD, dout), jnp.float32)] * 4,
    )(xp, xa, wsp, wnp, wsa, wna)


def _combine_mm4(nbp, nba, sp, sa, bp, ba, weights, din, dout):
    wsp, wnp, wsa, wna = weights
    bx = pl.BlockSpec((BLK, din), lambda i: (i, 0))
    bn = pl.BlockSpec((BLK, 128), lambda i: (i, 0))
    bb = pl.BlockSpec((1, din), lambda i: (0, 0))
    bw = pl.BlockSpec((din, dout), lambda i: (0, 0))
    bo = pl.BlockSpec((BLK, dout), lambda i: (i, 0))
    return pl.pallas_call(
        _combine_mm4_body,
        grid=(GRID,),
        in_specs=[bn, bn, bx, bx, bb, bb, bw, bw, bw, bw],
        out_specs=[bo, bo, bo, bo],
        out_shape=[jax.ShapeDtypeStruct((N_PAD, dout), jnp.float32)] * 4,
    )(nbp, nba, sp, sa, bp, ba, wsp, wnp, wsa, wna)


def _final_combine(nbp, nba, sp, sa, bp, ba, dout):
    bx = pl.BlockSpec((FBLK, dout), lambda i: (i, 0))
    bn = pl.BlockSpec((FBLK, 128), lambda i: (i, 0))
    bb = pl.BlockSpec((1, dout), lambda i: (0, 0))
    return pl.pallas_call(
        _final_body,
        grid=(FGRID,),
        in_specs=[bn, bn, bx, bx, bb, bb],
        out_specs=[bx, bx],
        out_shape=[jax.ShapeDtypeStruct((N, dout), jnp.float32)] * 2,
    )(nbp, nba, sp, sa, bp, ba)


def _agg_one(y_hbm, z_hbm, src_hbm, dst_hbm, out_hbm,
             src_v, dst_v, bufs, gsems, ssems, acc):
    sid = lax.axis_index("s")
    base = sid * ROWS_PER_TILE
    pltpu.sync_copy(z_hbm, acc.at[pl.ds(base, ROWS_PER_TILE)])
    plsc.subcore_barrier()

    def gwait(b):
        pltpu.make_async_copy(y_hbm.at[src_v.at[0]], bufs[b], gsems[b]).wait()

    def swait(b):
        pltpu.make_async_copy(bufs[b], acc.at[dst_v.at[0]], ssems[b]).wait()

    def seg_body(g, carry):
        pltpu.sync_copy(src_hbm.at[sid, pl.ds(g * SEG, SEG)], src_v)
        pltpu.sync_copy(dst_hbm.at[sid, pl.ds(g * SEG, SEG)], dst_v)
        for j in range(2):
            pltpu.async_copy(y_hbm.at[src_v.at[j]], bufs[j], gsems[j])
        for j in range(SEG):
            b = j % NBUF
            gwait(b)
            pltpu.async_copy(bufs[b], acc.at[dst_v.at[j]], ssems[b], add=True)
            if j + 2 < SEG:
                nb = (j + 2) % NBUF
                if j > 0:
                    swait(nb)
                pltpu.async_copy(y_hbm.at[src_v.at[j + 2]], bufs[nb], gsems[nb])
        for b in range(NBUF):
            swait(b)
        return carry

    lax.fori_loop(0, NCHUNK // SEG, seg_body, 0)
    plsc.subcore_barrier()
    pltpu.sync_copy(acc.at[pl.ds(base, ROWS_PER_TILE)],
                    out_hbm.at[pl.ds(base, ROWS_PER_TILE), pl.ds(0, acc.shape[1])])


def _make_sc_agg(dout):
    mesh = plsc.VectorSubcoreMesh(core_axis_name="c", subcore_axis_name="s",
                                  num_cores=NC, num_subcores=NS)

    @functools.partial(
        pl.kernel,
        out_type=[jax.ShapeDtypeStruct((N_PAD, 128), jnp.float32),
                  jax.ShapeDtypeStruct((N_PAD, 128), jnp.float32)],
        mesh=mesh,
        compiler_params=pltpu.CompilerParams(use_tc_tiling_on_sc=False),
        scratch_types=[
            pltpu.VMEM((SEG, CHUNK), jnp.int32),
            pltpu.VMEM((SEG, CHUNK), jnp.int32),
        ] + [pltpu.VMEM((CHUNK, dout), jnp.float32)] * NBUF
          + [pltpu.SemaphoreType.DMA] * (2 * NBUF)
          + [pltpu.VMEM_SHARED((N_PAD, dout), jnp.float32)],
    )
    def agg(yp, ya, zeros, src_pa, dst_pa, src_ap, dst_ap,
            outp, outa, src_v, dst_v, *rest):
        bufs = list(rest[:NBUF])
        gsems = list(rest[NBUF:2 * NBUF])
        ssems = list(rest[2 * NBUF:3 * NBUF])
        acc = rest[3 * NBUF]
        cid = lax.axis_index("c")

        @pl.when(cid == 0)
        def _():
            _agg_one(yp, zeros, src_pa, dst_pa, outp,
                     src_v, dst_v, bufs, gsems, ssems, acc)

        @pl.when(cid == 1)
        def _():
            _agg_one(ya, zeros, src_ap, dst_ap, outa,
                     src_v, dst_v, bufs, gsems, ssems, acc)

    return agg


_sc_agg = {d: _make_sc_agg(d) for d in (64, 16)}


def _prep_edges(edge):
    pad = NS * EDGES_PER_TILE - E
    src = jnp.pad(edge[1], (0, pad), constant_values=0)
    dst = jnp.pad(edge[0], (0, pad), constant_values=N)
    return (src.reshape(NS, NCHUNK, CHUNK), dst.reshape(NS, NCHUNK, CHUNK))


def kernel(ft_p, ft_a, edge_pa, edge_ap,
           Wself_p1, Wnb_p1, b_p1, Wself_a1, Wnb_a1, b_a1,
           Wself_p2, Wnb_p2, b_p2, Wself_a2, Wnb_a2, b_a2,
           Wself_p3, Wnb_p3, b_p3, Wself_a3, Wnb_a3, b_a3,
           Wself_p4, Wnb_p4, b_p4, Wself_a4, Wnb_a4, b_a4):
    layers = [
        (Wself_p1, Wnb_p1, b_p1, Wself_a1, Wnb_a1, b_a1),
        (Wself_p2, Wnb_p2, b_p2, Wself_a2, Wnb_a2, b_a2),
        (Wself_p3, Wnb_p3, b_p3, Wself_a3, Wnb_a3, b_a3),
        (Wself_p4, Wnb_p4, b_p4, Wself_a4, Wnb_a4, b_a4),
    ]
    xp = jnp.pad(ft_p, ((0, N_PAD - N), (0, 0)))
    xa = jnp.pad(ft_a, ((0, N_PAD - N), (0, 0)))
    src_pa, dst_pa = _prep_edges(edge_pa)
    src_ap, dst_ap = _prep_edges(edge_ap)

    nbp = nba = sp = sa = None
    for l, (wsp, wnp, bp, wsa, wna, ba) in enumerate(layers):
        din, dout = DIMS[l], DIMS[l + 1]
        if l == 0:
            sp, yp, sa, ya = _mm4(xp, xa, (wsp, wnp, wsa, wna), din, dout)
        else:
            bp_prev = layers[l - 1][2].reshape(1, din)
            ba_prev = layers[l - 1][5].reshape(1, din)
            sp, yp, sa, ya = _combine_mm4(nbp, nba, sp, sa, bp_prev, ba_prev,
                                          (wsp, wnp, wsa, wna), din, dout)
        zeros = jnp.zeros((ROWS_PER_TILE, dout), jnp.float32)
        nbp, nba = _sc_agg[dout](yp, ya, zeros,
                                 src_pa, dst_pa, src_ap, dst_ap)

    return _final_combine(nbp, nba, sp, sa,
                          b_p4.reshape(1, 16), b_a4.reshape(1, 16), 16)


# speedup vs baseline: 1.3688x; 1.0621x over previous
import functools

import jax
import jax.numpy as jnp
from jax import lax
from jax.experimental import pallas as pl
from jax.experimental.pallas import tpu as pltpu
from jax.experimental.pallas import tpu_sc as plsc

N = 25000
N_PAD = 25088
E = 400000
NS = 16
NC = 2
CHUNK = 128
EDGES_PER_TILE = 25088
NCHUNK = EDGES_PER_TILE // CHUNK
SEG = 14
NBUF = 3
ROWS_PER_TILE = N_PAD // NS
BLK = 1792
GRID = N_PAD // BLK
FBLK = 1000
FGRID = N // FBLK
DIMS = [128, 64, 64, 64, 16]


def _mm4_body(xp_r, xa_r, wsp_r, wnp_r, wsa_r, wna_r,
              sp_o, yp_o, sa_o, ya_o):
    xp = xp_r[...]
    xa = xa_r[...]
    sp_o[...] = jnp.dot(xp, wsp_r[...], preferred_element_type=jnp.float32)
    ya_o[...] = jnp.dot(xp, wna_r[...], preferred_element_type=jnp.float32)
    sa_o[...] = jnp.dot(xa, wsa_r[...], preferred_element_type=jnp.float32)
    yp_o[...] = jnp.dot(xa, wnp_r[...], preferred_element_type=jnp.float32)


def _combine_mm4_body(nbp_r, nba_r, sp_r, sa_r, bp_r, ba_r,
                      wsp_r, wnp_r, wsa_r, wna_r,
                      sp_o, yp_o, sa_o, ya_o):
    din = sp_r.shape[1]
    xp = jnp.maximum((sp_r[...] + nbp_r[:, :din]) * 0.5 + bp_r[...], 0.0)
    xa = jnp.maximum((sa_r[...] + nba_r[:, :din]) * 0.5 + ba_r[...], 0.0)
    sp_o[...] = jnp.dot(xp, wsp_r[...], preferred_element_type=jnp.float32)
    ya_o[...] = jnp.dot(xp, wna_r[...], preferred_element_type=jnp.float32)
    sa_o[...] = jnp.dot(xa, wsa_r[...], preferred_element_type=jnp.float32)
    yp_o[...] = jnp.dot(xa, wnp_r[...], preferred_element_type=jnp.float32)


def _final_body(nbp_r, nba_r, sp_r, sa_r, bp_r, ba_r, op_o, oa_o):
    dout = sp_r.shape[1]
    op_o[...] = (sp_r[...] + nbp_r[:, :dout]) * 0.5 + bp_r[...]
    oa_o[...] = (sa_r[...] + nba_r[:, :dout]) * 0.5 + ba_r[...]


def _mm4(xp, xa, weights, din, dout):
    wsp, wnp, wsa, wna = weights
    bx = pl.BlockSpec((FBLK, din), lambda i: (i, 0))
    bw = pl.BlockSpec((din, dout), lambda i: (0, 0))
    bo = pl.BlockSpec((FBLK, dout), lambda i: (i, 0))
    return pl.pallas_call(
        _mm4_body,
        grid=(FGRID,),
        in_specs=[bx, bx, bw, bw, bw, bw],
        out_specs=[bo, bo, bo, bo],
        out_shape=[jax.ShapeDtypeStruct((N_PAD, dout), jnp.float32)] * 4,
    )(xp, xa, wsp, wnp, wsa, wna)


def _combine_mm4(nbp, nba, sp, sa, bp, ba, weights, din, dout):
    wsp, wnp, wsa, wna = weights
    bx = pl.BlockSpec((BLK, din), lambda i: (i, 0))
    bn = pl.BlockSpec((BLK, 128), lambda i: (i, 0))
    bb = pl.BlockSpec((1, din), lambda i: (0, 0))
    bw = pl.BlockSpec((din, dout), lambda i: (0, 0))
    bo = pl.BlockSpec((BLK, dout), lambda i: (i, 0))
    return pl.pallas_call(
        _combine_mm4_body,
        grid=(GRID,),
        in_specs=[bn, bn, bx, bx, bb, bb, bw, bw, bw, bw],
        out_specs=[bo, bo, bo, bo],
        out_shape=[jax.ShapeDtypeStruct((N_PAD, dout), jnp.float32)] * 4,
    )(nbp, nba, sp, sa, bp, ba, wsp, wnp, wsa, wna)


def _final_combine(nbp, nba, sp, sa, bp, ba, dout):
    bx = pl.BlockSpec((FBLK, dout), lambda i: (i, 0))
    bn = pl.BlockSpec((FBLK, 128), lambda i: (i, 0))
    bb = pl.BlockSpec((1, dout), lambda i: (0, 0))
    return pl.pallas_call(
        _final_body,
        grid=(FGRID,),
        in_specs=[bn, bn, bx, bx, bb, bb],
        out_specs=[bx, bx],
        out_shape=[jax.ShapeDtypeStruct((N, dout), jnp.float32)] * 2,
    )(nbp, nba, sp, sa, bp, ba)


def _agg_one(y_hbm, z_hbm, src_hbm, dst_hbm, out_hbm,
             src_v, dst_v, bufs, gsems, ssems, acc):
    sid = lax.axis_index("s")
    base = sid * ROWS_PER_TILE
    pltpu.sync_copy(z_hbm, acc.at[pl.ds(base, ROWS_PER_TILE)])
    plsc.subcore_barrier()

    def gwait(b):
        pltpu.make_async_copy(y_hbm.at[src_v.at[0]], bufs[b], gsems[b]).wait()

    def swait(b):
        pltpu.make_async_copy(bufs[b], acc.at[dst_v.at[0]], ssems[b]).wait()

    def seg_body(g, carry):
        pltpu.sync_copy(src_hbm.at[sid, pl.ds(g * SEG, SEG)], src_v)
        pltpu.sync_copy(dst_hbm.at[sid, pl.ds(g * SEG, SEG)], dst_v)
        for j in range(2):
            pltpu.async_copy(y_hbm.at[src_v.at[j]], bufs[j], gsems[j])
        for j in range(SEG):
            b = j % NBUF
            gwait(b)
            pltpu.async_copy(bufs[b], acc.at[dst_v.at[j]], ssems[b], add=True)
            if j + 2 < SEG:
                nb = (j + 2) % NBUF
                if j > 0:
                    swait(nb)
                pltpu.async_copy(y_hbm.at[src_v.at[j + 2]], bufs[nb], gsems[nb])
        for b in range(NBUF):
            swait(b)
        return carry

    lax.fori_loop(0, NCHUNK // SEG, seg_body, 0)
    plsc.subcore_barrier()
    pltpu.sync_copy(acc.at[pl.ds(base, ROWS_PER_TILE)],
                    out_hbm.at[pl.ds(base, ROWS_PER_TILE), pl.ds(0, acc.shape[1])])


QUAD = 4
NQ = NCHUNK // QUAD
UPS = 7


def _agg_one_quad(y_hbm, z_hbm, src_hbm, dst_hbm, out_hbm,
                  src_v, dst_v, bufs, gsems, ssems, acc):
    sid = lax.axis_index("s")
    base = sid * ROWS_PER_TILE
    pltpu.sync_copy(z_hbm, acc.at[pl.ds(base, ROWS_PER_TILE)])
    plsc.subcore_barrier()

    def gwait(b):
        pltpu.make_async_copy(y_hbm.at[src_v.at[0]], bufs[b], gsems[b]).wait()

    def swait(b):
        pltpu.make_async_copy(bufs[b], acc.at[dst_v.at[0]], ssems[b]).wait()

    def seg_body(g, carry):
        pltpu.sync_copy(src_hbm.at[sid, pl.ds(g * UPS, UPS)], src_v)
        pltpu.sync_copy(dst_hbm.at[sid, pl.ds(g * UPS, UPS)], dst_v)
        for u in range(2):
            pltpu.async_copy(y_hbm.at[src_v.at[u]], bufs[u], gsems[u])
        for u in range(UPS):
            b = u % NBUF
            gwait(b)
            pltpu.async_copy(bufs[b], acc.at[dst_v.at[u]], ssems[b], add=True)
            if u + 2 < UPS:
                nb2 = (u + 2) % NBUF
                if u > 0:
                    swait(nb2)
                pltpu.async_copy(y_hbm.at[src_v.at[u + 2]], bufs[nb2], gsems[nb2])
        for b in range(NBUF):
            swait(b)
        return carry

    lax.fori_loop(0, NQ // UPS, seg_body, 0)
    plsc.subcore_barrier()
    pltpu.sync_copy(acc.at[pl.ds(base, ROWS_PER_TILE)],
                    out_hbm.at[pl.ds(base, ROWS_PER_TILE), pl.ds(0, acc.shape[1])])


def _make_sc_agg(dout):
    mesh = plsc.VectorSubcoreMesh(core_axis_name="c", subcore_axis_name="s",
                                  num_cores=NC, num_subcores=NS)

    quad = dout == 16
    idxshape = (UPS, QUAD * CHUNK) if quad else (SEG, CHUNK)
    bufshape = (QUAD * CHUNK, dout) if quad else (CHUNK, dout)
    body = _agg_one_quad if quad else _agg_one

    @functools.partial(
        pl.kernel,
        out_type=[jax.ShapeDtypeStruct((N_PAD, 128), jnp.float32),
                  jax.ShapeDtypeStruct((N_PAD, 128), jnp.float32)],
        mesh=mesh,
        compiler_params=pltpu.CompilerParams(use_tc_tiling_on_sc=False),
        scratch_types=[
            pltpu.VMEM(idxshape, jnp.int32),
            pltpu.VMEM(idxshape, jnp.int32),
        ] + [pltpu.VMEM(bufshape, jnp.float32)] * NBUF
          + [pltpu.SemaphoreType.DMA] * (2 * NBUF)
          + [pltpu.VMEM_SHARED((N_PAD, dout), jnp.float32)],
    )
    def agg(yp, ya, zeros, src_pa, dst_pa, src_ap, dst_ap,
            outp, outa, src_v, dst_v, *rest):
        bufs = list(rest[:NBUF])
        gsems = list(rest[NBUF:2 * NBUF])
        ssems = list(rest[2 * NBUF:3 * NBUF])
        acc = rest[3 * NBUF]
        cid = lax.axis_index("c")

        @pl.when(cid == 0)
        def _():
            body(yp, zeros, src_pa, dst_pa, outp,
                 src_v, dst_v, bufs, gsems, ssems, acc)

        @pl.when(cid == 1)
        def _():
            body(ya, zeros, src_ap, dst_ap, outa,
                 src_v, dst_v, bufs, gsems, ssems, acc)

    return agg


_sc_agg = {d: _make_sc_agg(d) for d in (64, 16)}


def _prep_edges(edge):
    pad = NS * EDGES_PER_TILE - E
    src = jnp.pad(edge[1], (0, pad), constant_values=0)
    dst = jnp.pad(edge[0], (0, pad), constant_values=N)
    return (src.reshape(NS, NCHUNK, CHUNK), dst.reshape(NS, NCHUNK, CHUNK))


def kernel(ft_p, ft_a, edge_pa, edge_ap,
           Wself_p1, Wnb_p1, b_p1, Wself_a1, Wnb_a1, b_a1,
           Wself_p2, Wnb_p2, b_p2, Wself_a2, Wnb_a2, b_a2,
           Wself_p3, Wnb_p3, b_p3, Wself_a3, Wnb_a3, b_a3,
           Wself_p4, Wnb_p4, b_p4, Wself_a4, Wnb_a4, b_a4):
    layers = [
        (Wself_p1, Wnb_p1, b_p1, Wself_a1, Wnb_a1, b_a1),
        (Wself_p2, Wnb_p2, b_p2, Wself_a2, Wnb_a2, b_a2),
        (Wself_p3, Wnb_p3, b_p3, Wself_a3, Wnb_a3, b_a3),
        (Wself_p4, Wnb_p4, b_p4, Wself_a4, Wnb_a4, b_a4),
    ]
    xp, xa = ft_p, ft_a
    src_pa, dst_pa = _prep_edges(edge_pa)
    src_ap, dst_ap = _prep_edges(edge_ap)

    nbp = nba = sp = sa = None
    for l, (wsp, wnp, bp, wsa, wna, ba) in enumerate(layers):
        din, dout = DIMS[l], DIMS[l + 1]
        if l == 0:
            sp, yp, sa, ya = _mm4(xp, xa, (wsp, wnp, wsa, wna), din, dout)
        else:
            bp_prev = layers[l - 1][2].reshape(1, din)
            ba_prev = layers[l - 1][5].reshape(1, din)
            sp, yp, sa, ya = _combine_mm4(nbp, nba, sp, sa, bp_prev, ba_prev,
                                          (wsp, wnp, wsa, wna), din, dout)
        zeros = jnp.zeros((ROWS_PER_TILE, dout), jnp.float32)
        if dout == 16:
            args = (src_pa.reshape(NS, NQ, QUAD * CHUNK),
                    dst_pa.reshape(NS, NQ, QUAD * CHUNK),
                    src_ap.reshape(NS, NQ, QUAD * CHUNK),
                    dst_ap.reshape(NS, NQ, QUAD * CHUNK))
        else:
            args = (src_pa, dst_pa, src_ap, dst_ap)
        nbp, nba = _sc_agg[dout](yp, ya, zeros, *args)

    return _final_combine(nbp, nba, sp, sa,
                          b_p4.reshape(1, 16), b_a4.reshape(1, 16), 16)


# speedup vs baseline: 1.4375x; 1.0501x over previous
import functools

import jax
import jax.numpy as jnp
from jax import lax
from jax.experimental import pallas as pl
from jax.experimental.pallas import tpu as pltpu
from jax.experimental.pallas import tpu_sc as plsc

N = 25000
N_PAD = 25088
E = 400000
NS = 16
NC = 2
CHUNK = 128
EDGES_PER_TILE = 25088
NCHUNK = EDGES_PER_TILE // CHUNK
SEG = 14
NBUF = 3
ROWS_PER_TILE = N_PAD // NS
BLK = 1792
GRID = N_PAD // BLK
FBLK = 1000
FGRID = N // FBLK
DIMS = [128, 64, 64, 64, 16]


def _mm4_body(xp_r, xa_r, wsp_r, wnp_r, wsa_r, wna_r,
              sp_o, yp_o, sa_o, ya_o):
    xp = xp_r[...]
    xa = xa_r[...]
    sp_o[...] = jnp.dot(xp, wsp_r[...], preferred_element_type=jnp.float32)
    ya_o[...] = jnp.dot(xp, wna_r[...], preferred_element_type=jnp.float32)
    sa_o[...] = jnp.dot(xa, wsa_r[...], preferred_element_type=jnp.float32)
    yp_o[...] = jnp.dot(xa, wnp_r[...], preferred_element_type=jnp.float32)


def _combine_mm4_body(nbp_r, nba_r, sp_r, sa_r, bp_r, ba_r,
                      wsp_r, wnp_r, wsa_r, wna_r,
                      sp_o, yp_o, sa_o, ya_o):
    din = sp_r.shape[1]
    xp = jnp.maximum((sp_r[...] + nbp_r[:, :din]) * 0.5 + bp_r[...], 0.0)
    xa = jnp.maximum((sa_r[...] + nba_r[:, :din]) * 0.5 + ba_r[...], 0.0)
    sp_o[...] = jnp.dot(xp, wsp_r[...], preferred_element_type=jnp.float32)
    ya_o[...] = jnp.dot(xp, wna_r[...], preferred_element_type=jnp.float32)
    sa_o[...] = jnp.dot(xa, wsa_r[...], preferred_element_type=jnp.float32)
    yp_o[...] = jnp.dot(xa, wnp_r[...], preferred_element_type=jnp.float32)


def _final_body(nbp_r, nba_r, sp_r, sa_r, bp_r, ba_r, op_o, oa_o):
    dout = sp_r.shape[1]
    op_o[...] = (sp_r[...] + nbp_r[:, :dout]) * 0.5 + bp_r[...]
    oa_o[...] = (sa_r[...] + nba_r[:, :dout]) * 0.5 + ba_r[...]


def _mm4(xp, xa, weights, din, dout):
    wsp, wnp, wsa, wna = weights
    bx = pl.BlockSpec((FBLK, din), lambda i: (i, 0))
    bw = pl.BlockSpec((din, dout), lambda i: (0, 0))
    bo = pl.BlockSpec((FBLK, dout), lambda i: (i, 0))
    return pl.pallas_call(
        _mm4_body,
        grid=(FGRID,),
        in_specs=[bx, bx, bw, bw, bw, bw],
        out_specs=[bo, bo, bo, bo],
        out_shape=[jax.ShapeDtypeStruct((N_PAD, dout), jnp.float32)] * 4,
    )(xp, xa, wsp, wnp, wsa, wna)


def _combine_mm4(nbp, nba, sp, sa, bp, ba, weights, din, dout):
    wsp, wnp, wsa, wna = weights
    bx = pl.BlockSpec((BLK, din), lambda i: (i, 0))
    bn = pl.BlockSpec((BLK, 128), lambda i: (i, 0))
    bb = pl.BlockSpec((1, din), lambda i: (0, 0))
    bw = pl.BlockSpec((din, dout), lambda i: (0, 0))
    bo = pl.BlockSpec((BLK, dout), lambda i: (i, 0))
    return pl.pallas_call(
        _combine_mm4_body,
        grid=(GRID,),
        in_specs=[bn, bn, bx, bx, bb, bb, bw, bw, bw, bw],
        out_specs=[bo, bo, bo, bo],
        out_shape=[jax.ShapeDtypeStruct((N_PAD, dout), jnp.float32)] * 4,
    )(nbp, nba, sp, sa, bp, ba, wsp, wnp, wsa, wna)


def _final_combine(nbp, nba, sp, sa, bp, ba, dout):
    bx = pl.BlockSpec((FBLK, dout), lambda i: (i, 0))
    bn = pl.BlockSpec((FBLK, 128), lambda i: (i, 0))
    bb = pl.BlockSpec((1, dout), lambda i: (0, 0))
    return pl.pallas_call(
        _final_body,
        grid=(FGRID,),
        in_specs=[bn, bn, bx, bx, bb, bb],
        out_specs=[bx, bx],
        out_shape=[jax.ShapeDtypeStruct((N, dout), jnp.float32)] * 2,
    )(nbp, nba, sp, sa, bp, ba)


def _agg_one(y_hbm, z_hbm, e_hbm, out_hbm,
             src_v, dst_v, bufs, gsems, ssems, acc):
    sid = lax.axis_index("s")
    base = sid * ROWS_PER_TILE
    pltpu.sync_copy(z_hbm, acc.at[pl.ds(base, ROWS_PER_TILE)])
    plsc.subcore_barrier()

    def gwait(b):
        pltpu.make_async_copy(y_hbm.at[src_v.at[0]], bufs[b], gsems[b]).wait()

    def swait(b):
        pltpu.make_async_copy(bufs[b], acc.at[dst_v.at[0]], ssems[b]).wait()

    def seg_body(g, carry):
        pltpu.sync_copy(e_hbm.at[1, sid, pl.ds(g * SEG, SEG)], src_v)
        pltpu.sync_copy(e_hbm.at[0, sid, pl.ds(g * SEG, SEG)], dst_v)
        for j in range(2):
            pltpu.async_copy(y_hbm.at[src_v.at[j]], bufs[j], gsems[j])
        for j in range(SEG):
            b = j % NBUF
            gwait(b)
            pltpu.async_copy(bufs[b], acc.at[dst_v.at[j]], ssems[b], add=True)
            if j + 2 < SEG:
                nb = (j + 2) % NBUF
                if j > 0:
                    swait(nb)
                pltpu.async_copy(y_hbm.at[src_v.at[j + 2]], bufs[nb], gsems[nb])
        for b in range(NBUF):
            swait(b)
        return carry

    lax.fori_loop(0, NCHUNK // SEG, seg_body, 0)
    plsc.subcore_barrier()
    pltpu.sync_copy(acc.at[pl.ds(base, ROWS_PER_TILE)],
                    out_hbm.at[pl.ds(base, ROWS_PER_TILE), pl.ds(0, acc.shape[1])])


QUAD = 4
NQ = NCHUNK // QUAD
UPS = 7


def _agg_one_quad(y_hbm, z_hbm, e_hbm, out_hbm,
                  src_v, dst_v, bufs, gsems, ssems, acc):
    sid = lax.axis_index("s")
    base = sid * ROWS_PER_TILE
    pltpu.sync_copy(z_hbm, acc.at[pl.ds(base, ROWS_PER_TILE)])
    plsc.subcore_barrier()

    def gwait(b):
        pltpu.make_async_copy(y_hbm.at[src_v.at[0]], bufs[b], gsems[b]).wait()

    def swait(b):
        pltpu.make_async_copy(bufs[b], acc.at[dst_v.at[0]], ssems[b]).wait()

    def seg_body(g, carry):
        pltpu.sync_copy(e_hbm.at[1, sid, pl.ds(g * UPS, UPS)], src_v)
        pltpu.sync_copy(e_hbm.at[0, sid, pl.ds(g * UPS, UPS)], dst_v)
        for u in range(2):
            pltpu.async_copy(y_hbm.at[src_v.at[u]], bufs[u], gsems[u])
        for u in range(UPS):
            b = u % NBUF
            gwait(b)
            pltpu.async_copy(bufs[b], acc.at[dst_v.at[u]], ssems[b], add=True)
            if u + 2 < UPS:
                nb2 = (u + 2) % NBUF
                if u > 0:
                    swait(nb2)
                pltpu.async_copy(y_hbm.at[src_v.at[u + 2]], bufs[nb2], gsems[nb2])
        for b in range(NBUF):
            swait(b)
        return carry

    lax.fori_loop(0, NQ // UPS, seg_body, 0)
    plsc.subcore_barrier()
    pltpu.sync_copy(acc.at[pl.ds(base, ROWS_PER_TILE)],
                    out_hbm.at[pl.ds(base, ROWS_PER_TILE), pl.ds(0, acc.shape[1])])


def _make_sc_agg(dout):
    mesh = plsc.VectorSubcoreMesh(core_axis_name="c", subcore_axis_name="s",
                                  num_cores=NC, num_subcores=NS)

    quad = dout == 16
    idxshape = (UPS, QUAD * CHUNK) if quad else (SEG, CHUNK)
    bufshape = (QUAD * CHUNK, dout) if quad else (CHUNK, dout)
    body = _agg_one_quad if quad else _agg_one

    @functools.partial(
        pl.kernel,
        out_type=[jax.ShapeDtypeStruct((N_PAD, 128), jnp.float32),
                  jax.ShapeDtypeStruct((N_PAD, 128), jnp.float32)],
        mesh=mesh,
        compiler_params=pltpu.CompilerParams(use_tc_tiling_on_sc=False),
        scratch_types=[
            pltpu.VMEM(idxshape, jnp.int32),
            pltpu.VMEM(idxshape, jnp.int32),
        ] + [pltpu.VMEM(bufshape, jnp.float32)] * NBUF
          + [pltpu.SemaphoreType.DMA] * (2 * NBUF)
          + [pltpu.VMEM_SHARED((N_PAD, dout), jnp.float32)],
    )
    def agg(yp, ya, zeros, epa, eap,
            outp, outa, src_v, dst_v, *rest):
        bufs = list(rest[:NBUF])
        gsems = list(rest[NBUF:2 * NBUF])
        ssems = list(rest[2 * NBUF:3 * NBUF])
        acc = rest[3 * NBUF]
        cid = lax.axis_index("c")

        @pl.when(cid == 0)
        def _():
            body(yp, zeros, epa, outp, src_v, dst_v, bufs, gsems, ssems, acc)

        @pl.when(cid == 1)
        def _():
            body(ya, zeros, eap, outa, src_v, dst_v, bufs, gsems, ssems, acc)

    return agg


_sc_agg = {d: _make_sc_agg(d) for d in (64, 16)}


def _prep_edges(edge):
    pad = NS * EDGES_PER_TILE - E
    padv = jnp.broadcast_to(jnp.array([[N], [0]], jnp.int32), (2, pad))
    return jnp.concatenate([edge, padv], axis=1).reshape(2, NS, NCHUNK, CHUNK)


def kernel(ft_p, ft_a, edge_pa, edge_ap,
           Wself_p1, Wnb_p1, b_p1, Wself_a1, Wnb_a1, b_a1,
           Wself_p2, Wnb_p2, b_p2, Wself_a2, Wnb_a2, b_a2,
           Wself_p3, Wnb_p3, b_p3, Wself_a3, Wnb_a3, b_a3,
           Wself_p4, Wnb_p4, b_p4, Wself_a4, Wnb_a4, b_a4):
    layers = [
        (Wself_p1, Wnb_p1, b_p1, Wself_a1, Wnb_a1, b_a1),
        (Wself_p2, Wnb_p2, b_p2, Wself_a2, Wnb_a2, b_a2),
        (Wself_p3, Wnb_p3, b_p3, Wself_a3, Wnb_a3, b_a3),
        (Wself_p4, Wnb_p4, b_p4, Wself_a4, Wnb_a4, b_a4),
    ]
    xp, xa = ft_p, ft_a
    epa = _prep_edges(edge_pa)
    eap = _prep_edges(edge_ap)

    nbp = nba = sp = sa = None
    for l, (wsp, wnp, bp, wsa, wna, ba) in enumerate(layers):
        din, dout = DIMS[l], DIMS[l + 1]
        if l == 0:
            sp, yp, sa, ya = _mm4(xp, xa, (wsp, wnp, wsa, wna), din, dout)
        else:
            bp_prev = layers[l - 1][2].reshape(1, din)
            ba_prev = layers[l - 1][5].reshape(1, din)
            sp, yp, sa, ya = _combine_mm4(nbp, nba, sp, sa, bp_prev, ba_prev,
                                          (wsp, wnp, wsa, wna), din, dout)
        zeros = jnp.zeros((ROWS_PER_TILE, dout), jnp.float32)
        if dout == 16:
            args = (epa.reshape(2, NS, NQ, QUAD * CHUNK),
                    eap.reshape(2, NS, NQ, QUAD * CHUNK))
        else:
            args = (epa, eap)
        nbp, nba = _sc_agg[dout](yp, ya, zeros, *args)

    return _final_combine(nbp, nba, sp, sa,
                          b_p4.reshape(1, 16), b_a4.reshape(1, 16), 16)
